# Initial kernel scaffold; baseline (speedup 1.0000x reference)
import jax
import jax.numpy as jnp
from jax.experimental import pallas as pl


def kernel(x, edge_index, edge_weight, W1, b1, W2, b2):
    raise NotImplementedError("write your pallas kernel here")


# speedup vs baseline: 3.2102x; 3.2102x over previous
import functools

import jax
import jax.numpy as jnp
from jax import lax
from jax.experimental import pallas as pl
from jax.experimental.pallas import tpu as pltpu
from jax.experimental.pallas import tpu_sc as plsc

_N = 10000
_E = 320000
_C = 128
_DEG = 5
_L = 16
_NW = 32
_NPAD = 10240
_RPT = _NPAD // 16
_EW = 10240
_EPAD = _EW * _NW
_K = 128
_NCHUNK = _EW // _K

_mesh = plsc.VectorSubcoreMesh(core_axis_name="c", subcore_axis_name="s")
_sc_params = pltpu.CompilerParams(needs_layout_passes=False)


def _wid():
    return lax.axis_index("s") * 2 + lax.axis_index("c")


def _deg_body(col_hbm, w_hbm, out_hbm, col_v, w_v, deg_v):
    w = _wid()
    base = w * _EW
    pltpu.sync_copy(col_hbm.at[pl.ds(base, _EW)], col_v)
    pltpu.sync_copy(w_hbm.at[pl.ds(base, _EW)], w_v)
    zeros = jnp.zeros((_L,), jnp.float32)

    def zb(i, carry):
        deg_v[pl.ds(i * _L, _L)] = zeros
        return carry

    lax.fori_loop(0, _NPAD // _L, zb, 0)

    def eb(i, carry):
        sl = pl.ds(i * _L, _L)
        plsc.addupdate_scatter(deg_v, [col_v[sl]], w_v[sl])
        return carry

    lax.fori_loop(0, _EW // _L, eb, 0)
    pltpu.sync_copy(deg_v, out_hbm.at[w])


_deg_call = pl.kernel(
    _deg_body,
    out_type=jax.ShapeDtypeStruct((_NW, _NPAD), jnp.float32),
    mesh=_mesh,
    scratch_types=[
        pltpu.VMEM((_EW,), jnp.int32),
        pltpu.VMEM((_EW,), jnp.float32),
        pltpu.VMEM((_NPAD,), jnp.float32),
    ],
    compiler_params=_sc_params,
)


def _prep_body(x_ref, degp_ref, W1_ref, b1_ref, W2_ref, b2_ref,
               dis_ref, coeffs_ref, res0_ref):
    deg = jnp.sum(degp_ref[...], axis=0)
    dis_ref[...] = jnp.minimum(lax.rsqrt(deg), 1e6)
    x = x_ref[...]
    xs = x[0:_N, :]
    x_mean_c = jnp.mean(xs, axis=0, keepdims=True)
    x_mean = jnp.mean(x_mean_c)
    var = jnp.sum((xs - x_mean) ** 2) / (_N * _C - 1)
    x_std = jnp.sqrt(var)
    W1 = W1_ref[...]
    h = jnp.sum(W1[:, 0:_C] * x_mean_c, axis=1)
    h = h + W1[:, _C] * x_mean + W1[:, _C + 1] * x_std
    h = h + W1[:, _C + 2] * jnp.float32(_N) + W1[:, _C + 3] * jnp.float32(_E)
    h = jnp.maximum(h + b1_ref[...], 0.0)
    logits = jnp.sum(W2_ref[...] * h[None, :], axis=1) + b2_ref[...]
    m = jnp.max(logits)
    p = jnp.exp(logits - m)
    coeffs = p / jnp.sum(p)
    coeffs_ref[...] = jnp.concatenate(
        [coeffs, jnp.zeros((8 - (_DEG + 1),), jnp.float32)])
    res0_ref[...] = x * coeffs[0]


def _prep_call(x_pad, deg_part, W1, b1, W2, b2):
    return pl.pallas_call(
        _prep_body,
        out_shape=(
            jax.ShapeDtypeStruct((_NPAD,), jnp.float32),
            jax.ShapeDtypeStruct((8,), jnp.float32),
            jax.ShapeDtypeStruct((_NPAD, _C), jnp.float32),
        ),
    )(x_pad, deg_part, W1, b1, W2, b2)


def _norm_body(dis_hbm, row_hbm, col_hbm, w_hbm, norm_hbm,
               dis_v, row_v, col_v, w_v, norm_v):
    w = _wid()
    base = w * _EW
    pltpu.sync_copy(dis_hbm, dis_v)
    pltpu.sync_copy(row_hbm.at[pl.ds(base, _EW)], row_v)
    pltpu.sync_copy(col_hbm.at[pl.ds(base, _EW)], col_v)
    pltpu.sync_copy(w_hbm.at[pl.ds(base, _EW)], w_v)

    def eb(i, carry):
        sl = pl.ds(i * _L, _L)
        dr = plsc.load_gather(dis_v, [row_v[sl]])
        dc = plsc.load_gather(dis_v, [col_v[sl]])
        norm_v[sl] = dr * w_v[sl] * dc
        return carry

    lax.fori_loop(0, _EW // _L, eb, 0)
    pltpu.sync_copy(norm_v, norm_hbm.at[pl.ds(base, _EW)])


_norm_call = pl.kernel(
    _norm_body,
    out_type=jax.ShapeDtypeStruct((_EPAD,), jnp.float32),
    mesh=_mesh,
    scratch_types=[
        pltpu.VMEM((_NPAD,), jnp.float32),
        pltpu.VMEM((_EW,), jnp.int32),
        pltpu.VMEM((_EW,), jnp.int32),
        pltpu.VMEM((_EW,), jnp.float32),
        pltpu.VMEM((_EW,), jnp.float32),
    ],
    compiler_params=_sc_params,
)


def _hop_body(table_hbm, row_hbm, col_hbm, norm_hbm, out_hbm,
              col_v, row_v, norm_v, rows_v, acc_sh):
    c = lax.axis_index("c")
    s = lax.axis_index("s")
    w = s * 2 + c
    base = w * _EW
    zeros = jnp.zeros((_L,), jnp.float32)

    def zb(j, carry):
        for l in range(_C // _L):
            rows_v[j, l * _L:(l + 1) * _L] = zeros
        return carry

    lax.fori_loop(0, _K, zb, 0)
    for j in range(_RPT // _K):
        pltpu.sync_copy(rows_v, acc_sh.at[pl.ds(s * _RPT + j * _K, _K)])
    plsc.subcore_barrier()

    def chunk(i, carry):
        cb = base + i * _K
        pltpu.sync_copy(col_hbm.at[pl.ds(cb, _K)], col_v)
        pltpu.sync_copy(row_hbm.at[pl.ds(cb, _K)], row_v)
        pltpu.sync_copy(norm_hbm.at[pl.ds(cb, _K)], norm_v)
        pltpu.sync_copy(table_hbm.at[col_v], rows_v)

        def sb(e, carry2):
            sc16 = plsc.load_gather(norm_v, [jnp.full((_L,), e, jnp.int32)])
            for l in range(_C // _L):
                sl = pl.ds(l * _L, _L)
                rows_v[e, sl] = rows_v[e, sl] * sc16
            return carry2

        lax.fori_loop(0, _K, sb, 0)
        pltpu.sync_copy(rows_v, acc_sh.at[row_v], add=True)
        return carry

    lax.fori_loop(0, _NCHUNK, chunk, 0)
    plsc.subcore_barrier()
    pltpu.sync_copy(acc_sh.at[pl.ds(s * _RPT, _RPT)],
                    out_hbm.at[c, pl.ds(s * _RPT, _RPT)])


_hop_call = pl.kernel(
    _hop_body,
    out_type=jax.ShapeDtypeStruct((2, _NPAD, _C), jnp.float32),
    mesh=_mesh,
    scratch_types=[
        pltpu.VMEM((_K,), jnp.int32),
        pltpu.VMEM((_K,), jnp.int32),
        pltpu.VMEM((_K,), jnp.float32),
        pltpu.VMEM((_K, _C), jnp.float32),
        pltpu.VMEM_SHARED((_NPAD, _C), jnp.float32),
    ],
    compiler_params=_sc_params,
)


def _comb_body(p_ref, res_ref, cf_ref, tx_ref, resout_ref):
    txb = p_ref[0] + p_ref[1]
    tx_ref[...] = txb
    resout_ref[...] = res_ref[...] + cf_ref[0] * txb


def _comb_call(partial, res, cf):
    return pl.pallas_call(
        _comb_body,
        out_shape=(
            jax.ShapeDtypeStruct((_NPAD, _C), jnp.float32),
            jax.ShapeDtypeStruct((_NPAD, _C), jnp.float32),
        ),
    )(partial, res, cf)


def kernel(x, edge_index, edge_weight, W1, b1, W2, b2):
    row = jnp.pad(edge_index[0], (0, _EPAD - _E))
    col = jnp.pad(edge_index[1], (0, _EPAD - _E))
    w = jnp.pad(edge_weight, (0, _EPAD - _E))
    x_pad = jnp.pad(x, ((0, _NPAD - _N), (0, 0)))

    deg_part = _deg_call(col, w)
    dis, coeffs, res = _prep_call(x_pad, deg_part, W1, b1, W2, b2)
    norm = _norm_call(dis, row, col, w)

    tx = x_pad
    for k in range(1, _DEG + 1):
        partial = _hop_call(tx, row, col, norm)
        tx, res = _comb_call(partial, res, coeffs[k:k + 1])
    return res[:_N]


# speedup vs baseline: 4.5408x; 1.4145x over previous
import functools

import jax
import jax.numpy as jnp
from jax import lax
from jax.experimental import pallas as pl
from jax.experimental.pallas import tpu as pltpu
from jax.experimental.pallas import tpu_sc as plsc

_N = 10000
_E = 320000
_C = 128
_DEG = 5
_L = 16
_NW = 32
_NPAD = 10240
_RPT = _NPAD // 16
_EW = 10240
_EPAD = _EW * _NW
_K = 64
_NCHUNK = _EW // _K

_mesh = plsc.VectorSubcoreMesh(core_axis_name="c", subcore_axis_name="s")
_sc_params = pltpu.CompilerParams(needs_layout_passes=False)


def _wid():
    return lax.axis_index("s") * 2 + lax.axis_index("c")


def _deg_body(col_hbm, w_hbm, out_hbm, col_v, w_v, deg_v):
    w = _wid()
    base = w * _EW
    pltpu.sync_copy(col_hbm.at[pl.ds(base, _EW)], col_v)
    pltpu.sync_copy(w_hbm.at[pl.ds(base, _EW)], w_v)
    zeros = jnp.zeros((_L,), jnp.float32)

    def zb(i, carry):
        deg_v[pl.ds(i * _L, _L)] = zeros
        return carry

    lax.fori_loop(0, _NPAD // _L, zb, 0)

    def eb(i, carry):
        sl = pl.ds(i * _L, _L)
        plsc.addupdate_scatter(deg_v, [col_v[sl]], w_v[sl])
        return carry

    lax.fori_loop(0, _EW // _L, eb, 0)
    pltpu.sync_copy(deg_v, out_hbm.at[w])


_deg_call = pl.kernel(
    _deg_body,
    out_type=jax.ShapeDtypeStruct((_NW, _NPAD), jnp.float32),
    mesh=_mesh,
    scratch_types=[
        pltpu.VMEM((_EW,), jnp.int32),
        pltpu.VMEM((_EW,), jnp.float32),
        pltpu.VMEM((_NPAD,), jnp.float32),
    ],
    compiler_params=_sc_params,
)


def _prep_body(x_ref, degp_ref, W1_ref, b1_ref, W2_ref, b2_ref,
               dis_ref, coeffs_ref, res0_ref):
    deg = jnp.sum(degp_ref[...], axis=0)
    dis_ref[...] = jnp.minimum(lax.rsqrt(deg), 1e6)
    x = x_ref[...]
    xs = x[0:_N, :]
    x_mean_c = jnp.mean(xs, axis=0, keepdims=True)
    x_mean = jnp.mean(x_mean_c)
    var = jnp.sum((xs - x_mean) ** 2) / (_N * _C - 1)
    x_std = jnp.sqrt(var)
    W1 = W1_ref[...]
    h = jnp.sum(W1[:, 0:_C] * x_mean_c, axis=1)
    h = h + W1[:, _C] * x_mean + W1[:, _C + 1] * x_std
    h = h + W1[:, _C + 2] * jnp.float32(_N) + W1[:, _C + 3] * jnp.float32(_E)
    h = jnp.maximum(h + b1_ref[...], 0.0)
    logits = jnp.sum(W2_ref[...] * h[None, :], axis=1) + b2_ref[...]
    m = jnp.max(logits)
    p = jnp.exp(logits - m)
    coeffs = p / jnp.sum(p)
    coeffs_ref[...] = jnp.concatenate(
        [coeffs, jnp.zeros((8 - (_DEG + 1),), jnp.float32)])
    res0_ref[...] = x * coeffs[0]


def _prep_call(x_pad, deg_part, W1, b1, W2, b2):
    return pl.pallas_call(
        _prep_body,
        out_shape=(
            jax.ShapeDtypeStruct((_NPAD,), jnp.float32),
            jax.ShapeDtypeStruct((8,), jnp.float32),
            jax.ShapeDtypeStruct((_NPAD, _C), jnp.float32),
        ),
    )(x_pad, deg_part, W1, b1, W2, b2)


def _norm_body(dis_hbm, row_hbm, col_hbm, w_hbm, norm_hbm,
               dis_v, row_v, col_v, w_v, norm_v):
    w = _wid()
    base = w * _EW
    pltpu.sync_copy(dis_hbm, dis_v)
    pltpu.sync_copy(row_hbm.at[pl.ds(base, _EW)], row_v)
    pltpu.sync_copy(col_hbm.at[pl.ds(base, _EW)], col_v)
    pltpu.sync_copy(w_hbm.at[pl.ds(base, _EW)], w_v)

    def eb(i, carry):
        sl = pl.ds(i * _L, _L)
        dr = plsc.load_gather(dis_v, [row_v[sl]])
        dc = plsc.load_gather(dis_v, [col_v[sl]])
        norm_v[sl] = dr * w_v[sl] * dc
        return carry

    lax.fori_loop(0, _EW // _L, eb, 0)
    pltpu.sync_copy(norm_v, norm_hbm.at[pl.ds(base, _EW)])


_norm_call = pl.kernel(
    _norm_body,
    out_type=jax.ShapeDtypeStruct((_EPAD,), jnp.float32),
    mesh=_mesh,
    scratch_types=[
        pltpu.VMEM((_NPAD,), jnp.float32),
        pltpu.VMEM((_EW,), jnp.int32),
        pltpu.VMEM((_EW,), jnp.int32),
        pltpu.VMEM((_EW,), jnp.float32),
        pltpu.VMEM((_EW,), jnp.float32),
    ],
    compiler_params=_sc_params,
)


_NBUF = 4
_NPACK = 8
_NJ8 = _NCHUNK // _NPACK


def _hop_body(table_hbm, packed_hbm, out_hbm,
              b0, b1, b2, b3, p0, p1, p2, p3, p4, p5, p6, p7, acc_sh,
              g0, g1, g2, g3, s0, s1, s2, s3,
              q0, q1, q2, q3, q4, q5, q6, q7):
    c = lax.axis_index("c")
    s = lax.axis_index("s")
    w = s * 2 + c
    cw = w * _NCHUNK
    bufs = (b0, b1, b2, b3)
    pbufs = (p0, p1, p2, p3, p4, p5, p6, p7)
    gsems = (g0, g1, g2, g3)
    ssems = (s0, s1, s2, s3)
    psems = (q0, q1, q2, q3, q4, q5, q6, q7)

    for i in range(_NPACK):
        pltpu.async_copy(packed_hbm.at[cw + i], pbufs[i], psems[i])

    zeros = jnp.zeros((_L,), jnp.float32)

    def zb(j, carry):
        for l in range(_C // _L):
            b0[j, l * _L:(l + 1) * _L] = zeros
        return carry

    lax.fori_loop(0, _K, zb, 0)
    for j in range(_RPT // _K):
        pltpu.sync_copy(b0, acc_sh.at[pl.ds(s * _RPT + j * _K, _K)])
    plsc.subcore_barrier()

    def wait_pack(ci, p):
        pltpu.make_async_copy(packed_hbm.at[cw + ci], pbufs[p], psems[p]).wait()

    def issue_gather(p, b):
        pltpu.async_copy(table_hbm.at[pbufs[p].at[0]], bufs[b], gsems[b])

    def wait_gather(p, b):
        pltpu.make_async_copy(
            table_hbm.at[pbufs[p].at[0]], bufs[b], gsems[b]).wait()

    def issue_scatter(p, b):
        pltpu.async_copy(bufs[b], acc_sh.at[pbufs[p].at[1]], ssems[b],
                         add=True)

    def wait_scatter(p, b):
        pltpu.make_async_copy(
            bufs[b], acc_sh.at[pbufs[p].at[1]], ssems[b]).wait()

    def scale(b, p):
        buf = bufs[b]
        pbuf = pbufs[p]
        two = jnp.full((_L,), 2, jnp.int32)

        def sb(e, carry):
            ei = jnp.full((_L,), e, jnp.int32)
            sc16 = plsc.bitcast(plsc.load_gather(pbuf, [two, ei]), jnp.float32)
            for l in range(_C // _L):
                sl = pl.ds(l * _L, _L)
                buf[e, sl] = buf[e, sl] * sc16
            return carry

        lax.fori_loop(0, _K, sb, 0)

    for q in range(_NBUF):
        wait_pack(q, q)
        issue_gather(q, q)

    def body(j, carry):
        for t in range(_NPACK):
            ci = j * _NPACK + t
            b = t % 4
            wait_gather(t, b)
            scale(b, t)
            issue_scatter(t, b)
            bq = (t + 2) % 4
            pp2 = (t + 2) % _NPACK
            ppn = (t + 6) % _NPACK

            def _refill_a():
                wait_scatter(ppn, bq)

            def _refill_b():
                pltpu.async_copy(packed_hbm.at[cw + ci + 6], pbufs[ppn],
                                 psems[ppn])

            def _refill_c():
                wait_pack(ci + 2, pp2)
                issue_gather(pp2, bq)

            if t < 2:
                @pl.when(j >= 1)
                def _():
                    _refill_a()
                    _refill_b()
                    _refill_c()
            elif t < 6:
                _refill_a()

                @pl.when(j < _NJ8 - 1)
                def _():
                    _refill_b()
                _refill_c()
            else:
                _refill_a()

                @pl.when(j < _NJ8 - 1)
                def _():
                    _refill_b()
                    _refill_c()
        return carry

    lax.fori_loop(0, _NJ8, body, 0)
    wait_scatter(6, 2)
    wait_scatter(7, 3)
    plsc.subcore_barrier()
    pltpu.sync_copy(acc_sh.at[pl.ds(s * _RPT, _RPT)],
                    out_hbm.at[c, pl.ds(s * _RPT, _RPT)])


_hop_call = pl.kernel(
    _hop_body,
    out_type=jax.ShapeDtypeStruct((2, _NPAD, _C), jnp.float32),
    mesh=_mesh,
    scratch_types=(
        [pltpu.VMEM((_K, _C), jnp.float32)] * _NBUF
        + [pltpu.VMEM((3, _K), jnp.int32)] * _NPACK
        + [pltpu.VMEM_SHARED((_NPAD, _C), jnp.float32)]
        + [pltpu.SemaphoreType.DMA] * (_NBUF + _NBUF + _NPACK)
    ),
    compiler_params=_sc_params,
)


def _comb_body(p_ref, res_ref, cf_ref, tx_ref, resout_ref):
    txb = p_ref[0] + p_ref[1]
    tx_ref[...] = txb
    resout_ref[...] = res_ref[...] + cf_ref[0] * txb


def _comb_call(partial, res, cf):
    return pl.pallas_call(
        _comb_body,
        out_shape=(
            jax.ShapeDtypeStruct((_NPAD, _C), jnp.float32),
            jax.ShapeDtypeStruct((_NPAD, _C), jnp.float32),
        ),
    )(partial, res, cf)


def kernel(x, edge_index, edge_weight, W1, b1, W2, b2):
    row = jnp.pad(edge_index[0], (0, _EPAD - _E))
    col = jnp.pad(edge_index[1], (0, _EPAD - _E))
    w = jnp.pad(edge_weight, (0, _EPAD - _E))
    x_pad = jnp.pad(x, ((0, _NPAD - _N), (0, 0)))

    deg_part = _deg_call(col, w)
    dis, coeffs, res = _prep_call(x_pad, deg_part, W1, b1, W2, b2)
    norm = _norm_call(dis, row, col, w)

    packed = jnp.stack(
        [col.reshape(_EPAD // _K, _K),
         row.reshape(_EPAD // _K, _K),
         lax.bitcast_convert_type(norm, jnp.int32).reshape(_EPAD // _K, _K)],
        axis=1)
    tx = x_pad
    for k in range(1, _DEG + 1):
        partial = _hop_call(tx, packed)
        tx, res = _comb_call(partial, res, coeffs[k:k + 1])
    return res[:_N]


# speedup vs baseline: 5.5823x; 1.2294x over previous
import functools

import jax
import jax.numpy as jnp
from jax import lax
from jax.experimental import pallas as pl
from jax.experimental.pallas import tpu as pltpu
from jax.experimental.pallas import tpu_sc as plsc

_N = 10000
_E = 320000
_C = 128
_DEG = 5
_L = 16
_NW = 32
_NPAD = 10240
_RPT = _NPAD // 16
_EW = 10240
_EPAD = _EW * _NW
_K = 64
_NCHUNK = _EW // _K

_mesh = plsc.VectorSubcoreMesh(core_axis_name="c", subcore_axis_name="s")
_sc_params = pltpu.CompilerParams(needs_layout_passes=False)


def _wid():
    return lax.axis_index("s") * 2 + lax.axis_index("c")


def _deg_body(col_hbm, w_hbm, out_hbm, col_v, w_v, deg_v):
    w = _wid()
    base = w * _EW
    pltpu.sync_copy(col_hbm.at[pl.ds(base, _EW)], col_v)
    pltpu.sync_copy(w_hbm.at[pl.ds(base, _EW)], w_v)
    zeros = jnp.zeros((_L,), jnp.float32)

    def zb(i, carry):
        deg_v[pl.ds(i * _L, _L)] = zeros
        return carry

    lax.fori_loop(0, _NPAD // _L, zb, 0)

    def eb(i, carry):
        sl = pl.ds(i * _L, _L)
        plsc.addupdate_scatter(deg_v, [col_v[sl]], w_v[sl])
        return carry

    lax.fori_loop(0, _EW // _L, eb, 0)
    pltpu.sync_copy(deg_v, out_hbm.at[w])


_deg_call = pl.kernel(
    _deg_body,
    out_type=jax.ShapeDtypeStruct((_NW, _NPAD), jnp.float32),
    mesh=_mesh,
    scratch_types=[
        pltpu.VMEM((_EW,), jnp.int32),
        pltpu.VMEM((_EW,), jnp.float32),
        pltpu.VMEM((_NPAD,), jnp.float32),
    ],
    compiler_params=_sc_params,
)


def _prep_body(x_ref, degp_ref, W1_ref, b1_ref, W2_ref, b2_ref,
               dis_ref, coeffs_ref, res0_ref):
    deg = jnp.sum(degp_ref[...], axis=0)
    dis_ref[...] = jnp.minimum(lax.rsqrt(deg), 1e6)
    x = x_ref[...]
    xs = x[0:_N, :]
    x_mean_c = jnp.mean(xs, axis=0, keepdims=True)
    x_mean = jnp.mean(x_mean_c)
    var = jnp.sum((xs - x_mean) ** 2) / (_N * _C - 1)
    x_std = jnp.sqrt(var)
    W1 = W1_ref[...]
    h = jnp.sum(W1[:, 0:_C] * x_mean_c, axis=1)
    h = h + W1[:, _C] * x_mean + W1[:, _C + 1] * x_std
    h = h + W1[:, _C + 2] * jnp.float32(_N) + W1[:, _C + 3] * jnp.float32(_E)
    h = jnp.maximum(h + b1_ref[...], 0.0)
    logits = jnp.sum(W2_ref[...] * h[None, :], axis=1) + b2_ref[...]
    m = jnp.max(logits)
    p = jnp.exp(logits - m)
    coeffs = p / jnp.sum(p)
    coeffs_ref[...] = jnp.concatenate(
        [coeffs, jnp.zeros((8 - (_DEG + 1),), jnp.float32)])
    res0_ref[...] = x * coeffs[0]


def _prep_call(x_pad, deg_part, W1, b1, W2, b2):
    return pl.pallas_call(
        _prep_body,
        out_shape=(
            jax.ShapeDtypeStruct((_NPAD,), jnp.float32),
            jax.ShapeDtypeStruct((8,), jnp.float32),
            jax.ShapeDtypeStruct((_NPAD, _C), jnp.float32),
        ),
    )(x_pad, deg_part, W1, b1, W2, b2)


def _norm_body(dis_hbm, row_hbm, col_hbm, w_hbm, norm_hbm,
               dis_v, row_v, col_v, w_v, norm_v):
    w = _wid()
    base = w * _EW
    pltpu.sync_copy(dis_hbm, dis_v)
    pltpu.sync_copy(row_hbm.at[pl.ds(base, _EW)], row_v)
    pltpu.sync_copy(col_hbm.at[pl.ds(base, _EW)], col_v)
    pltpu.sync_copy(w_hbm.at[pl.ds(base, _EW)], w_v)

    def eb(i, carry):
        sl = pl.ds(i * _L, _L)
        dr = plsc.load_gather(dis_v, [row_v[sl]])
        dc = plsc.load_gather(dis_v, [col_v[sl]])
        norm_v[sl] = dr * w_v[sl] * dc
        return carry

    lax.fori_loop(0, _EW // _L, eb, 0)
    pltpu.sync_copy(norm_v, norm_hbm.at[pl.ds(base, _EW)])


_norm_call = pl.kernel(
    _norm_body,
    out_type=jax.ShapeDtypeStruct((_EPAD,), jnp.float32),
    mesh=_mesh,
    scratch_types=[
        pltpu.VMEM((_NPAD,), jnp.float32),
        pltpu.VMEM((_EW,), jnp.int32),
        pltpu.VMEM((_EW,), jnp.int32),
        pltpu.VMEM((_EW,), jnp.float32),
        pltpu.VMEM((_EW,), jnp.float32),
    ],
    compiler_params=_sc_params,
)


_NBUF = 4
_NPACK = 8
_M0 = 240
_M1 = 2 * _NCHUNK - _M0


def _hop_body(table_hbm, packed_hbm, out_hbm,
              b0, b1, b2, b3, p0, p1, p2, p3, p4, p5, p6, p7, acc_sh,
              g0, g1, g2, g3, s0, s1, s2, s3,
              q0, q1, q2, q3, q4, q5, q6, q7):
    c = lax.axis_index("c")
    s = lax.axis_index("s")
    cw = s * (2 * _NCHUNK) + c * _M0
    nj = jnp.where(c == 0, _M0 // _NPACK, _M1 // _NPACK)
    bufs = (b0, b1, b2, b3)
    pbufs = (p0, p1, p2, p3, p4, p5, p6, p7)
    gsems = (g0, g1, g2, g3)
    ssems = (s0, s1, s2, s3)
    psems = (q0, q1, q2, q3, q4, q5, q6, q7)

    for i in range(_NPACK):
        pltpu.async_copy(packed_hbm.at[cw + i], pbufs[i], psems[i])

    zeros = jnp.zeros((_L,), jnp.float32)

    def zb(j, carry):
        for l in range(_C // _L):
            b0[j, l * _L:(l + 1) * _L] = zeros
        return carry

    lax.fori_loop(0, _K, zb, 0)
    for j in range(_RPT // _K):
        pltpu.sync_copy(b0, acc_sh.at[pl.ds(s * _RPT + j * _K, _K)])
    plsc.subcore_barrier()

    def wait_pack(ci, p):
        pltpu.make_async_copy(packed_hbm.at[cw + ci], pbufs[p], psems[p]).wait()

    def issue_gather(p, b):
        pltpu.async_copy(table_hbm.at[pbufs[p].at[0]], bufs[b], gsems[b])

    def wait_gather(p, b):
        pltpu.make_async_copy(
            table_hbm.at[pbufs[p].at[0]], bufs[b], gsems[b]).wait()

    def issue_scatter(p, b):
        pltpu.async_copy(bufs[b], acc_sh.at[pbufs[p].at[1]], ssems[b],
                         add=True)

    def wait_scatter(p, b):
        pltpu.make_async_copy(
            bufs[b], acc_sh.at[pbufs[p].at[1]], ssems[b]).wait()

    def scale(b, p):
        buf = bufs[b]
        pbuf = pbufs[p]
        two = jnp.full((_L,), 2, jnp.int32)

        def sb(e, carry):
            ei = jnp.full((_L,), e, jnp.int32)
            sc16 = plsc.bitcast(plsc.load_gather(pbuf, [two, ei]), jnp.float32)
            for l in range(_C // _L):
                sl = pl.ds(l * _L, _L)
                buf[e, sl] = buf[e, sl] * sc16
            return carry

        lax.fori_loop(0, _K, sb, 0)

    for q in range(_NBUF):
        wait_pack(q, q)
        issue_gather(q, q)

    def body(j, carry):
        for t in range(_NPACK):
            ci = j * _NPACK + t
            b = t % 4
            wait_gather(t, b)
            scale(b, t)
            issue_scatter(t, b)
            bq = (t + 2) % 4
            pp2 = (t + 2) % _NPACK
            ppn = (t + 6) % _NPACK

            def _refill_a():
                wait_scatter(ppn, bq)

            def _refill_b():
                pltpu.async_copy(packed_hbm.at[cw + ci + 6], pbufs[ppn],
                                 psems[ppn])

            def _refill_c():
                wait_pack(ci + 2, pp2)
                issue_gather(pp2, bq)

            if t < 2:
                @pl.when(j >= 1)
                def _():
                    _refill_a()
                    _refill_b()
                    _refill_c()
            elif t < 6:
                _refill_a()

                @pl.when(j < nj - 1)
                def _():
                    _refill_b()
                _refill_c()
            else:
                _refill_a()

                @pl.when(j < nj - 1)
                def _():
                    _refill_b()
                    _refill_c()
        return carry

    lax.fori_loop(0, nj, body, 0)
    wait_scatter(6, 2)
    wait_scatter(7, 3)
    plsc.subcore_barrier()
    pltpu.sync_copy(acc_sh.at[pl.ds(s * _RPT, _RPT)],
                    out_hbm.at[c, pl.ds(s * _RPT, _RPT)])


_hop_call = pl.kernel(
    _hop_body,
    out_type=jax.ShapeDtypeStruct((2, _NPAD, _C), jnp.float32),
    mesh=_mesh,
    scratch_types=(
        [pltpu.VMEM((_K, _C), jnp.float32)] * _NBUF
        + [pltpu.VMEM((3, _K), jnp.int32)] * _NPACK
        + [pltpu.VMEM_SHARED((_NPAD, _C), jnp.float32)]
        + [pltpu.SemaphoreType.DMA] * (_NBUF + _NBUF + _NPACK)
    ),
    compiler_params=_sc_params,
)


def _comb_body(p_ref, res_ref, cf_ref, tx_ref, resout_ref):
    txb = p_ref[0] + p_ref[1]
    tx_ref[...] = txb
    resout_ref[...] = res_ref[...] + cf_ref[0] * txb


def _comb_call(partial, res, cf):
    return pl.pallas_call(
        _comb_body,
        out_shape=(
            jax.ShapeDtypeStruct((_NPAD, _C), jnp.float32),
            jax.ShapeDtypeStruct((_NPAD, _C), jnp.float32),
        ),
    )(partial, res, cf)


def kernel(x, edge_index, edge_weight, W1, b1, W2, b2):
    row = jnp.pad(edge_index[0], (0, _EPAD - _E))
    col = jnp.pad(edge_index[1], (0, _EPAD - _E))
    w = jnp.pad(edge_weight, (0, _EPAD - _E))
    x_pad = jnp.pad(x, ((0, _NPAD - _N), (0, 0)))

    deg_part = _deg_call(col, w)
    dis, coeffs, res = _prep_call(x_pad, deg_part, W1, b1, W2, b2)
    norm = _norm_call(dis, row, col, w)

    packed = jnp.stack(
        [col.reshape(_EPAD // _K, _K),
         row.reshape(_EPAD // _K, _K),
         lax.bitcast_convert_type(norm, jnp.int32).reshape(_EPAD // _K, _K)],
        axis=1)
    tx = x_pad
    for k in range(1, _DEG + 1):
        partial = _hop_call(tx, packed)
        tx, res = _comb_call(partial, res, coeffs[k:k + 1])
    return res[:_N]
